# baseline (speedup 1.0000x reference)
import functools

import jax
import jax.numpy as jnp
from jax import lax
from jax.experimental import pallas as pl
from jax.experimental.pallas import tpu as pltpu
from jax.experimental.pallas import tpu_sc as plsc

_N = 10000
_NPAD = 10240
_E = 320000
_NC = 2
_NS = 16
_NW = _NC * _NS
_CH = 128
_KA = 80
_KB = 80
_TOT = _NS * (_KA + _KB)
_G = 8
_EPAD = _TOT * _CH
_K = _KA + _KB
_RPT = _NPAD // _NS
_D = 128
_DOUT = 64
_BLK = 512
_GRID = _NPAD // _BLK


def _mesh():
    return plsc.VectorSubcoreMesh(core_axis_name="c", subcore_axis_name="s",
                                  num_cores=_NC, num_subcores=_NS)


_KMAX = max(_KA, _KB)


def _make_agg(d):
    out_type = [jax.ShapeDtypeStruct((_NC, _NPAD, d), jnp.float32)]
    scratch = [
        pltpu.VMEM((_KMAX, _CH), jnp.int32),
        pltpu.VMEM((_KMAX, _CH), jnp.int32),
        pltpu.VMEM((_CH, d), jnp.float32),
        pltpu.VMEM_SHARED((_NPAD, d), jnp.float32),
        pltpu.SemaphoreType.DMA,
    ]

    def body(h_hbm, src_hbm, dst_hbm, zacc_hbm, out_hbm,
             src_v, dst_v, rows_v, acc_s, sem_g):
        c = lax.axis_index("c")
        s = lax.axis_index("s")
        r0 = s * _RPT
        wstart = jnp.where(c == 0, s * _KA, _NS * _KA + s * _KB)
        kb = jnp.where(c == 0, _KA, _KB)
        base = jnp.minimum(wstart, _TOT - _KMAX)
        off = wstart - base
        pltpu.sync_copy(src_hbm.at[pl.ds(base, _KMAX)], src_v)
        pltpu.sync_copy(dst_hbm.at[pl.ds(base, _KMAX)], dst_v)
        pltpu.sync_copy(zacc_hbm, acc_s.at[pl.ds(r0, _RPT)])
        plsc.subcore_barrier()

        def step(j, carry):
            jj = off + j
            pltpu.async_copy(h_hbm.at[src_v.at[jj]], rows_v,
                             sem_g).wait()
            pltpu.sync_copy(rows_v, acc_s.at[dst_v.at[jj]], add=True)
            return carry

        lax.fori_loop(0, kb, step, 0)
        plsc.subcore_barrier()
        pltpu.sync_copy(acc_s.at[pl.ds(r0, _RPT)],
                        out_hbm.at[c, pl.ds(r0, _RPT)])

    return pl.kernel(body, out_type=out_type, mesh=_mesh(),
                     scratch_types=scratch)


def _make_deg():
    out_type = [jax.ShapeDtypeStruct((_NPAD, _D), jnp.float32)] * 2
    _KD = _TOT // _NS
    scratch = [
        pltpu.VMEM((_KD, _CH), jnp.int32),
        pltpu.VMEM((_CH, _D), jnp.float32),
        pltpu.VMEM_SHARED((_NPAD, _D), jnp.float32),
        pltpu.SemaphoreType.DMA,
    ]

    def body(src_hbm, dst_hbm, zacc_hbm, ones_hbm, dego_hbm, degi_hbm,
             idx_v, ones_v, deg_s, sem):
        c = lax.axis_index("c")
        s = lax.axis_index("s")
        r0 = s * _RPT

        @pl.when(c == 0)
        def _stage_src():
            pltpu.sync_copy(src_hbm.at[pl.ds(s * _KD, _KD)], idx_v)

        @pl.when(c == 1)
        def _stage_dst():
            pltpu.sync_copy(dst_hbm.at[pl.ds(s * _KD, _KD)], idx_v)

        pltpu.sync_copy(ones_hbm, ones_v)
        pltpu.sync_copy(zacc_hbm, deg_s.at[pl.ds(r0, _RPT)])
        plsc.subcore_barrier()

        def step(j, carry):
            @pl.when(j >= 8)
            def _drain():
                pltpu.make_async_copy(ones_v, deg_s.at[idx_v.at[j]],
                                      sem).wait()
            pltpu.async_copy(ones_v, deg_s.at[idx_v.at[j]], sem,
                             add=True)
            return carry

        lax.fori_loop(0, _KD, step, 0)
        for _ in range(8):
            pltpu.make_async_copy(ones_v, deg_s.at[idx_v.at[0]],
                                  sem).wait()
        plsc.subcore_barrier()

        @pl.when(c == 0)
        def _out_src():
            pltpu.sync_copy(deg_s.at[pl.ds(r0, _RPT)],
                            dego_hbm.at[pl.ds(r0, _RPT)])

        @pl.when(c == 1)
        def _out_dst():
            pltpu.sync_copy(deg_s.at[pl.ds(r0, _RPT)],
                            degi_hbm.at[pl.ds(r0, _RPT)])

    return pl.kernel(body, out_type=out_type, mesh=_mesh(),
                     scratch_types=scratch)


def _spec_rows(d):
    return pl.BlockSpec((_BLK, d), lambda i: (i, 0))


def _spec_part(p, d):
    return pl.BlockSpec((1, _BLK, d), lambda i, _p=p: (_p, i, 0))


def _spec_full(r, c):
    return pl.BlockSpec((r, c), lambda i: (0, 0))


def _rsq(d_ref):
    return lax.rsqrt(jnp.maximum(d_ref[:, :1], 1.0))


def _tc_mm(x, w):
    def body(x_ref, w_ref, o_ref):
        o_ref[...] = jnp.dot(x_ref[...], w_ref[...],
                             preferred_element_type=jnp.float32)
    return pl.pallas_call(
        body, grid=(_GRID,),
        in_specs=[_spec_rows(_D), _spec_full(_D, _D)],
        out_specs=_spec_rows(_D),
        out_shape=jax.ShapeDtypeStruct((_NPAD, _D), jnp.float32),
    )(x, w)


def _tc_l2(p1, dego, wh):
    def body(pa, pb, dg, w_ref, o_ref):
        x2 = jnp.maximum(pa[0] + pb[0], 0.0) * _rsq(dg)
        o_ref[...] = jnp.dot(x2, w_ref[...],
                             preferred_element_type=jnp.float32)
    return pl.pallas_call(
        body, grid=(_GRID,),
        in_specs=[_spec_part(0, _D), _spec_part(1, _D),
                  _spec_rows(_D), _spec_full(_D, _D)],
        out_specs=_spec_rows(_D),
        out_shape=jax.ShapeDtypeStruct((_NPAD, _D), jnp.float32),
    )(p1, p1, dego, wh)


def _tc_l3(p2, dego, degi, w2):
    def body(pa, pb, do_, di_, w_ref, o_ref):
        x3 = jnp.maximum((pa[0] + pb[0]) * _rsq(di_), 0.0)
        o_ref[...] = jnp.dot(x3 * _rsq(do_), w_ref[...],
                             preferred_element_type=jnp.float32)
    return pl.pallas_call(
        body, grid=(_GRID,),
        in_specs=[_spec_part(0, _D), _spec_part(1, _D),
                  _spec_rows(_D), _spec_rows(_D),
                  _spec_full(_D, _D)],
        out_specs=_spec_rows(_D),
        out_shape=jax.ShapeDtypeStruct((_NPAD, _D), jnp.float32),
    )(p2, p2, dego, degi, w2)


def _tc_l4(p3, degi, b2):
    def body(pa, pb, di_, b_ref, o_ref):
        y = (pa[0, :, :_DOUT] + pb[0, :, :_DOUT]) * _rsq(di_) + b_ref[...]
        m = jnp.max(y, axis=1, keepdims=True)
        z = y - m
        o_ref[...] = z - jnp.log(jnp.sum(jnp.exp(z), axis=1, keepdims=True))
    return pl.pallas_call(
        body, grid=(_GRID,),
        in_specs=[_spec_part(0, _D), _spec_part(1, _D),
                  _spec_rows(_D), _spec_full(1, _DOUT)],
        out_specs=_spec_rows(_DOUT),
        out_shape=jax.ShapeDtypeStruct((_NPAD, _DOUT), jnp.float32),
    )(p3, p3, degi, b2)


_deg = _make_deg()
_agg128 = _make_agg(_D)


def kernel(features, edge_index, W1, Wh, W2, b2):
    f32 = jnp.float32
    x = jnp.zeros((_NPAD, _D), f32).at[:_N].set(features)
    pad = jnp.full((2, _EPAD - _E), _N, jnp.int32)
    ei = jnp.concatenate([edge_index.astype(jnp.int32), pad], axis=1)
    src = ei[0].reshape(_TOT, _CH)
    dst = ei[1].reshape(_TOT, _CH)
    zacc = jnp.zeros((_RPT, _D), f32)
    ones = jnp.ones((_CH, _D), f32)
    w2p = jnp.zeros((_D, _D), f32).at[:, :_DOUT].set(W2)

    h1 = _tc_mm(x, W1)
    dego, degi = _deg(src, dst, zacc, ones)
    (p1,) = _agg128(h1, src, dst, zacc)
    h2 = _tc_l2(p1, dego, Wh)
    (p2,) = _agg128(h2, src, dst, zacc)
    h3 = _tc_l3(p2, dego, degi, w2p)
    (p3,) = _agg128(h3, src, dst, zacc)
    y = _tc_l4(p3, degi, b2.reshape(1, _DOUT))
    return y[:_N]


# speedup vs baseline: 1.0907x; 1.0907x over previous
import functools

import jax
import jax.numpy as jnp
from jax import lax
from jax.experimental import pallas as pl
from jax.experimental.pallas import tpu as pltpu
from jax.experimental.pallas import tpu_sc as plsc

_N = 10000
_NPAD = 10240
_E = 320000
_NC = 2
_NS = 16
_NW = _NC * _NS
_CH = 128
_KA = 104
_KB = 56
_TOT = _NS * (_KA + _KB)
_G = 8
_EPAD = _TOT * _CH
_K = _KA + _KB
_RPT = _NPAD // _NS
_D = 128
_DOUT = 64
_BLK = 512
_GRID = _NPAD // _BLK


def _mesh():
    return plsc.VectorSubcoreMesh(core_axis_name="c", subcore_axis_name="s",
                                  num_cores=_NC, num_subcores=_NS)


_KMAX = max(_KA, _KB)


def _make_agg(d):
    out_type = [jax.ShapeDtypeStruct((_NC, _NPAD, d), jnp.float32)]
    scratch = [
        pltpu.VMEM((_KMAX, _CH), jnp.int32),
        pltpu.VMEM((_KMAX, _CH), jnp.int32),
        pltpu.VMEM((_CH, d), jnp.float32),
        pltpu.VMEM_SHARED((_NPAD, d), jnp.float32),
        pltpu.SemaphoreType.DMA,
    ]

    def body(h_hbm, src_hbm, dst_hbm, zacc_hbm, out_hbm,
             src_v, dst_v, rows_v, acc_s, sem_g):
        c = lax.axis_index("c")
        s = lax.axis_index("s")
        r0 = s * _RPT
        wstart = jnp.where(c == 0, s * _KA, _NS * _KA + s * _KB)
        kb = jnp.where(c == 0, _KA, _KB)
        base = jnp.minimum(wstart, _TOT - _KMAX)
        off = wstart - base
        pltpu.sync_copy(src_hbm.at[pl.ds(base, _KMAX)], src_v)
        pltpu.sync_copy(dst_hbm.at[pl.ds(base, _KMAX)], dst_v)
        pltpu.sync_copy(zacc_hbm, acc_s.at[pl.ds(r0, _RPT)])
        plsc.subcore_barrier()

        def step(j, carry):
            jj = off + j
            pltpu.async_copy(h_hbm.at[src_v.at[jj]], rows_v,
                             sem_g).wait()
            pltpu.sync_copy(rows_v, acc_s.at[dst_v.at[jj]], add=True)
            return carry

        lax.fori_loop(0, kb, step, 0)
        plsc.subcore_barrier()
        pltpu.sync_copy(acc_s.at[pl.ds(r0, _RPT)],
                        out_hbm.at[c, pl.ds(r0, _RPT)])

    return pl.kernel(body, out_type=out_type, mesh=_mesh(),
                     scratch_types=scratch)


def _make_deg():
    out_type = [jax.ShapeDtypeStruct((_NPAD, _D), jnp.float32)] * 2
    _KD = _TOT // _NS
    scratch = [
        pltpu.VMEM((_KD, _CH), jnp.int32),
        pltpu.VMEM((_CH, _D), jnp.float32),
        pltpu.VMEM_SHARED((_NPAD, _D), jnp.float32),
        pltpu.SemaphoreType.DMA,
    ]

    def body(src_hbm, dst_hbm, zacc_hbm, ones_hbm, dego_hbm, degi_hbm,
             idx_v, ones_v, deg_s, sem):
        c = lax.axis_index("c")
        s = lax.axis_index("s")
        r0 = s * _RPT

        @pl.when(c == 0)
        def _stage_src():
            pltpu.sync_copy(src_hbm.at[pl.ds(s * _KD, _KD)], idx_v)

        @pl.when(c == 1)
        def _stage_dst():
            pltpu.sync_copy(dst_hbm.at[pl.ds(s * _KD, _KD)], idx_v)

        pltpu.sync_copy(ones_hbm, ones_v)
        pltpu.sync_copy(zacc_hbm, deg_s.at[pl.ds(r0, _RPT)])
        plsc.subcore_barrier()

        def step(j, carry):
            @pl.when(j >= 8)
            def _drain():
                pltpu.make_async_copy(ones_v, deg_s.at[idx_v.at[j]],
                                      sem).wait()
            pltpu.async_copy(ones_v, deg_s.at[idx_v.at[j]], sem,
                             add=True)
            return carry

        lax.fori_loop(0, _KD, step, 0)
        for _ in range(8):
            pltpu.make_async_copy(ones_v, deg_s.at[idx_v.at[0]],
                                  sem).wait()
        plsc.subcore_barrier()

        @pl.when(c == 0)
        def _out_src():
            pltpu.sync_copy(deg_s.at[pl.ds(r0, _RPT)],
                            dego_hbm.at[pl.ds(r0, _RPT)])

        @pl.when(c == 1)
        def _out_dst():
            pltpu.sync_copy(deg_s.at[pl.ds(r0, _RPT)],
                            degi_hbm.at[pl.ds(r0, _RPT)])

    return pl.kernel(body, out_type=out_type, mesh=_mesh(),
                     scratch_types=scratch)


def _spec_rows(d):
    return pl.BlockSpec((_BLK, d), lambda i: (i, 0))


def _spec_part(p, d):
    return pl.BlockSpec((1, _BLK, d), lambda i, _p=p: (_p, i, 0))


def _spec_full(r, c):
    return pl.BlockSpec((r, c), lambda i: (0, 0))


def _rsq(d_ref):
    return lax.rsqrt(jnp.maximum(d_ref[:, :1], 1.0))


def _tc_mm(x, w):
    def body(x_ref, w_ref, o_ref):
        o_ref[...] = jnp.dot(x_ref[...], w_ref[...],
                             preferred_element_type=jnp.float32)
    return pl.pallas_call(
        body, grid=(_GRID,),
        in_specs=[_spec_rows(_D), _spec_full(_D, _D)],
        out_specs=_spec_rows(_D),
        out_shape=jax.ShapeDtypeStruct((_NPAD, _D), jnp.float32),
    )(x, w)


def _tc_l2(p1, dego, wh):
    def body(pa, pb, dg, w_ref, o_ref):
        x2 = jnp.maximum(pa[0] + pb[0], 0.0) * _rsq(dg)
        o_ref[...] = jnp.dot(x2, w_ref[...],
                             preferred_element_type=jnp.float32)
    return pl.pallas_call(
        body, grid=(_GRID,),
        in_specs=[_spec_part(0, _D), _spec_part(1, _D),
                  _spec_rows(_D), _spec_full(_D, _D)],
        out_specs=_spec_rows(_D),
        out_shape=jax.ShapeDtypeStruct((_NPAD, _D), jnp.float32),
    )(p1, p1, dego, wh)


def _tc_l3(p2, dego, degi, w2):
    def body(pa, pb, do_, di_, w_ref, o_ref):
        x3 = jnp.maximum((pa[0] + pb[0]) * _rsq(di_), 0.0)
        o_ref[...] = jnp.dot(x3 * _rsq(do_), w_ref[...],
                             preferred_element_type=jnp.float32)
    return pl.pallas_call(
        body, grid=(_GRID,),
        in_specs=[_spec_part(0, _D), _spec_part(1, _D),
                  _spec_rows(_D), _spec_rows(_D),
                  _spec_full(_D, _D)],
        out_specs=_spec_rows(_D),
        out_shape=jax.ShapeDtypeStruct((_NPAD, _D), jnp.float32),
    )(p2, p2, dego, degi, w2)


def _tc_l4(p3, degi, b2):
    def body(pa, pb, di_, b_ref, o_ref):
        y = (pa[0, :, :_DOUT] + pb[0, :, :_DOUT]) * _rsq(di_) + b_ref[...]
        m = jnp.max(y, axis=1, keepdims=True)
        z = y - m
        o_ref[...] = z - jnp.log(jnp.sum(jnp.exp(z), axis=1, keepdims=True))
    return pl.pallas_call(
        body, grid=(_GRID,),
        in_specs=[_spec_part(0, _D), _spec_part(1, _D),
                  _spec_rows(_D), _spec_full(1, _DOUT)],
        out_specs=_spec_rows(_DOUT),
        out_shape=jax.ShapeDtypeStruct((_NPAD, _DOUT), jnp.float32),
    )(p3, p3, degi, b2)


_deg = _make_deg()
_agg128 = _make_agg(_D)


def kernel(features, edge_index, W1, Wh, W2, b2):
    f32 = jnp.float32
    x = jnp.zeros((_NPAD, _D), f32).at[:_N].set(features)
    pad = jnp.full((2, _EPAD - _E), _N, jnp.int32)
    ei = jnp.concatenate([edge_index.astype(jnp.int32), pad], axis=1)
    src = ei[0].reshape(_TOT, _CH)
    dst = ei[1].reshape(_TOT, _CH)
    zacc = jnp.zeros((_RPT, _D), f32)
    ones = jnp.ones((_CH, _D), f32)
    w2p = jnp.zeros((_D, _D), f32).at[:, :_DOUT].set(W2)

    h1 = _tc_mm(x, W1)
    dego, degi = _deg(src, dst, zacc, ones)
    (p1,) = _agg128(h1, src, dst, zacc)
    h2 = _tc_l2(p1, dego, Wh)
    (p2,) = _agg128(h2, src, dst, zacc)
    h3 = _tc_l3(p2, dego, degi, w2p)
    (p3,) = _agg128(h3, src, dst, zacc)
    y = _tc_l4(p3, degi, b2.reshape(1, _DOUT))
    return y[:_N]


# speedup vs baseline: 1.1018x; 1.0102x over previous
import functools

import jax
import jax.numpy as jnp
from jax import lax
from jax.experimental import pallas as pl
from jax.experimental.pallas import tpu as pltpu
from jax.experimental.pallas import tpu_sc as plsc

_N = 10000
_NPAD = 10240
_E = 320000
_NC = 2
_NS = 16
_NW = _NC * _NS
_CH = 128
_KA = 104
_KB = 56
_TOT = _NS * (_KA + _KB)
_G = 8
_EPAD = _TOT * _CH
_K = _KA + _KB
_RPT = _NPAD // _NS
_D = 128
_DOUT = 64
_BLK = 512
_GRID = _NPAD // _BLK


def _mesh():
    return plsc.VectorSubcoreMesh(core_axis_name="c", subcore_axis_name="s",
                                  num_cores=_NC, num_subcores=_NS)


_KMAX = max(_KA, _KB)


def _make_agg(d):
    out_type = [jax.ShapeDtypeStruct((_NC, _NPAD, d), jnp.float32)]
    scratch = [
        pltpu.VMEM((_KMAX, _CH), jnp.int32),
        pltpu.VMEM((_KMAX, _CH), jnp.int32),
        pltpu.VMEM((_CH, d), jnp.float32),
        pltpu.VMEM_SHARED((_NPAD, d), jnp.float32),
        pltpu.SemaphoreType.DMA,
    ]

    def body(h_hbm, src_hbm, dst_hbm, zacc_hbm, out_hbm,
             src_v, dst_v, rows_v, acc_s, sem_g):
        c = lax.axis_index("c")
        s = lax.axis_index("s")
        w = c * _NS + s
        r0 = s * _RPT
        pltpu.sync_copy(src_hbm.at[w], src_v)
        pltpu.sync_copy(dst_hbm.at[w], dst_v)
        pltpu.sync_copy(zacc_hbm, acc_s.at[pl.ds(r0, _RPT)])
        plsc.subcore_barrier()

        def step(j, carry):
            pltpu.async_copy(h_hbm.at[src_v.at[j]], rows_v,
                             sem_g).wait()
            pltpu.sync_copy(rows_v, acc_s.at[dst_v.at[j]], add=True)
            return carry

        lax.fori_loop(0, _KB, step, 0)

        @pl.when(c == 0)
        def _tail():
            lax.fori_loop(0, _KA - _KB,
                          lambda j, cr: step(j + _KB, cr), 0)

        plsc.subcore_barrier()
        pltpu.sync_copy(acc_s.at[pl.ds(r0, _RPT)],
                        out_hbm.at[c, pl.ds(r0, _RPT)])

    return pl.kernel(body, out_type=out_type, mesh=_mesh(),
                     scratch_types=scratch)


def _make_deg():
    out_type = [jax.ShapeDtypeStruct((_NPAD, _D), jnp.float32)] * 2
    scratch = [
        pltpu.VMEM((2, _KMAX, _CH), jnp.int32),
        pltpu.VMEM((_CH, _D), jnp.float32),
        pltpu.VMEM_SHARED((_NPAD, _D), jnp.float32),
        pltpu.SemaphoreType.DMA,
    ]

    def body(src_hbm, dst_hbm, zacc_hbm, ones_hbm, dego_hbm, degi_hbm,
             idx_v, ones_v, deg_s, sem):
        c = lax.axis_index("c")
        s = lax.axis_index("s")
        r0 = s * _RPT

        @pl.when(c == 0)
        def _stage_src():
            pltpu.sync_copy(src_hbm.at[s], idx_v.at[0])
            pltpu.sync_copy(src_hbm.at[s + _NS], idx_v.at[1])

        @pl.when(c == 1)
        def _stage_dst():
            pltpu.sync_copy(dst_hbm.at[s], idx_v.at[0])
            pltpu.sync_copy(dst_hbm.at[s + _NS], idx_v.at[1])

        pltpu.sync_copy(ones_hbm, ones_v)
        pltpu.sync_copy(zacc_hbm, deg_s.at[pl.ds(r0, _RPT)])
        plsc.subcore_barrier()

        def step(j, carry):
            a = jnp.where(j >= _KA, 1, 0)
            b = jnp.where(j >= _KA, j - _KA, j)

            @pl.when(j >= 8)
            def _drain():
                pltpu.make_async_copy(ones_v, deg_s.at[idx_v.at[a, b]],
                                      sem).wait()
            pltpu.async_copy(ones_v, deg_s.at[idx_v.at[a, b]], sem,
                             add=True)
            return carry

        lax.fori_loop(0, _KA + _KB, step, 0)
        for _ in range(8):
            pltpu.make_async_copy(ones_v, deg_s.at[idx_v.at[0, 0]],
                                  sem).wait()
        plsc.subcore_barrier()

        @pl.when(c == 0)
        def _out_src():
            pltpu.sync_copy(deg_s.at[pl.ds(r0, _RPT)],
                            dego_hbm.at[pl.ds(r0, _RPT)])

        @pl.when(c == 1)
        def _out_dst():
            pltpu.sync_copy(deg_s.at[pl.ds(r0, _RPT)],
                            degi_hbm.at[pl.ds(r0, _RPT)])

    return pl.kernel(body, out_type=out_type, mesh=_mesh(),
                     scratch_types=scratch)


def _spec_rows(d):
    return pl.BlockSpec((_BLK, d), lambda i: (i, 0))


def _spec_part(p, d):
    return pl.BlockSpec((1, _BLK, d), lambda i, _p=p: (_p, i, 0))


def _spec_full(r, c):
    return pl.BlockSpec((r, c), lambda i: (0, 0))


def _rsq(d_ref):
    return lax.rsqrt(jnp.maximum(d_ref[:, :1], 1.0))


def _tc_mm(x, w):
    def body(x_ref, w_ref, o_ref):
        o_ref[...] = jnp.dot(x_ref[...], w_ref[...],
                             preferred_element_type=jnp.float32)
    return pl.pallas_call(
        body, grid=(_GRID,),
        in_specs=[_spec_rows(_D), _spec_full(_D, _D)],
        out_specs=_spec_rows(_D),
        out_shape=jax.ShapeDtypeStruct((_NPAD, _D), jnp.float32),
    )(x, w)


def _tc_l2(p1, dego, wh):
    def body(pa, pb, dg, w_ref, o_ref):
        x2 = jnp.maximum(pa[0] + pb[0], 0.0) * _rsq(dg)
        o_ref[...] = jnp.dot(x2, w_ref[...],
                             preferred_element_type=jnp.float32)
    return pl.pallas_call(
        body, grid=(_GRID,),
        in_specs=[_spec_part(0, _D), _spec_part(1, _D),
                  _spec_rows(_D), _spec_full(_D, _D)],
        out_specs=_spec_rows(_D),
        out_shape=jax.ShapeDtypeStruct((_NPAD, _D), jnp.float32),
    )(p1, p1, dego, wh)


def _tc_l3(p2, dego, degi, w2):
    def body(pa, pb, do_, di_, w_ref, o_ref):
        x3 = jnp.maximum((pa[0] + pb[0]) * _rsq(di_), 0.0)
        o_ref[...] = jnp.dot(x3 * _rsq(do_), w_ref[...],
                             preferred_element_type=jnp.float32)
    return pl.pallas_call(
        body, grid=(_GRID,),
        in_specs=[_spec_part(0, _D), _spec_part(1, _D),
                  _spec_rows(_D), _spec_rows(_D),
                  _spec_full(_D, _D)],
        out_specs=_spec_rows(_D),
        out_shape=jax.ShapeDtypeStruct((_NPAD, _D), jnp.float32),
    )(p2, p2, dego, degi, w2)


def _tc_l4(p3, degi, b2):
    def body(pa, pb, di_, b_ref, o_ref):
        y = (pa[0, :, :_DOUT] + pb[0, :, :_DOUT]) * _rsq(di_) + b_ref[...]
        m = jnp.max(y, axis=1, keepdims=True)
        z = y - m
        o_ref[...] = z - jnp.log(jnp.sum(jnp.exp(z), axis=1, keepdims=True))
    return pl.pallas_call(
        body, grid=(_GRID,),
        in_specs=[_spec_part(0, _D), _spec_part(1, _D),
                  _spec_rows(_D), _spec_full(1, _DOUT)],
        out_specs=_spec_rows(_DOUT),
        out_shape=jax.ShapeDtypeStruct((_NPAD, _DOUT), jnp.float32),
    )(p3, p3, degi, b2)


_deg = _make_deg()
_agg128 = _make_agg(_D)


def kernel(features, edge_index, W1, Wh, W2, b2):
    f32 = jnp.float32
    x = jnp.zeros((_NPAD, _D), f32).at[:_N].set(features)
    pad = jnp.full((2, _EPAD - _E), _N, jnp.int32)
    ei = jnp.concatenate([edge_index.astype(jnp.int32), pad], axis=1)

    def _layout(flat):
        c0 = flat[:_NS * _KA].reshape(_NS, _KA, _CH)
        c1 = flat[_NS * _KA:].reshape(_NS, _KB, _CH)
        c0 = jnp.pad(c0, ((0, 0), (0, _KMAX - _KA), (0, 0)),
                     constant_values=_N)
        c1 = jnp.pad(c1, ((0, 0), (0, _KMAX - _KB), (0, 0)),
                     constant_values=_N)
        return jnp.concatenate([c0, c1], axis=0)

    src = _layout(ei[0].reshape(_TOT, _CH))
    dst = _layout(ei[1].reshape(_TOT, _CH))
    zacc = jnp.zeros((_RPT, _D), f32)
    ones = jnp.ones((_CH, _D), f32)
    w2p = jnp.zeros((_D, _D), f32).at[:, :_DOUT].set(W2)

    h1 = _tc_mm(x, W1)
    dego, degi = _deg(src, dst, zacc, ones)
    (p1,) = _agg128(h1, src, dst, zacc)
    h2 = _tc_l2(p1, dego, Wh)
    (p2,) = _agg128(h2, src, dst, zacc)
    h3 = _tc_l3(p2, dego, degi, w2p)
    (p3,) = _agg128(h3, src, dst, zacc)
    y = _tc_l4(p3, degi, b2.reshape(1, _DOUT))
    return y[:_N]
